# baseline (speedup 1.0000x reference)
import functools

import jax
import jax.numpy as jnp
from jax import lax
from jax.experimental import pallas as pl
from jax.experimental.pallas import tpu as pltpu
from jax.experimental.pallas import tpu_sc as plsc

_N_CLS = 1024
_SEQ = 77
_N_CTX = 16
_CTX_DIM = 512
_NUM_CORES = 2
_NUM_SUBCORES = 16
_NW = _NUM_CORES * _NUM_SUBCORES
_CHUNK = 32
_NCHUNK = _N_CLS // _CHUNK
_NPOS = _SEQ - _N_CTX
_NGU = _NPOS * _NCHUNK
_NU = _NGU // _NW
_NBUF = 6
_CTX_ROWS = 32


def _assemble_body(tokg_hbm, ctx32_hbm, table_hbm, out_hbm,
                   st0, st1, st2, st3, st4, st5, ctx_rep, idx_all,
                   g0, g1, g2, g3, g4, g5, s0, s1, s2, s3, s4, s5, csem):
    wid = lax.axis_index("s") * _NUM_CORES + lax.axis_index("c")
    base = wid * _NU

    stages = (st0, st1, st2, st3, st4, st5)
    gsems = (g0, g1, g2, g3, g4, g5)
    ssems = (s0, s1, s2, s3, s4, s5)

    def unit_out(u):
        q = u // _NCHUNK
        t = jnp.where(q == 0, 0, q + _N_CTX)
        c0 = (u % _NCHUNK) * _CHUNK
        return t, c0

    def fire_gather(i, k):
        pltpu.async_copy(
            table_hbm.at[idx_all.at[pl.ds(i * _CHUNK, _CHUNK)]],
            stages[k], gsems[k])

    def wait_gather(k):
        pltpu.make_async_copy(
            table_hbm.at[pl.ds(0, _CHUNK)], stages[k], gsems[k]).wait()

    def fire_store(i, k):
        t, c0 = unit_out(base + i)
        pltpu.async_copy(
            stages[k], out_hbm.at[t, pl.ds(c0, _CHUNK)], ssems[k])

    def wait_store(k):
        pltpu.make_async_copy(
            stages[k], out_hbm.at[0, pl.ds(0, _CHUNK)], ssems[k]).wait()

    pltpu.sync_copy(tokg_hbm.at[pl.ds(base * _CHUNK, _NU * _CHUNK)], idx_all)
    fire_gather(0, 0)
    fire_gather(1, 1)
    fire_gather(2, 2)

    tctx = wid // 2
    cbase = (wid % 2) * (_N_CLS // 2)
    pltpu.sync_copy(
        ctx32_hbm.at[pl.ds(tctx * _CTX_ROWS, _CTX_ROWS)], ctx_rep)
    for k in range(_N_CLS // 2 // _CTX_ROWS):
        pltpu.async_copy(
            ctx_rep,
            out_hbm.at[tctx + 1, pl.ds(cbase + k * _CTX_ROWS, _CTX_ROWS)],
            csem)

    def body(ip, carry):
        for k in range(_NBUF):
            u = _NBUF * ip + k

            @pl.when(u + 3 < _NU)
            def _(u=u, k=k):
                kn = (k + 3) % _NBUF
                if k >= 3:
                    wait_store(kn)
                else:
                    @pl.when(ip > 0)
                    def _():
                        wait_store(kn)
                fire_gather(u + 3, kn)

            @pl.when(u < _NU)
            def _(u=u, k=k):
                wait_gather(k)
                fire_store(u, k)
        return carry

    lax.fori_loop(0, -(-_NU // _NBUF), body, 0)
    for k in range(_NBUF):
        wait_store(k)
    for _k in range(_N_CLS // 2 // _CTX_ROWS):
        pltpu.make_async_copy(
            ctx_rep, out_hbm.at[0, pl.ds(0, _CTX_ROWS)], csem).wait()


@jax.jit
def _assemble(tokg, ctx32, token_embedding):
    mesh = plsc.VectorSubcoreMesh(
        core_axis_name="c", subcore_axis_name="s",
        num_cores=_NUM_CORES, num_subcores=_NUM_SUBCORES)
    return pl.kernel(
        _assemble_body,
        out_type=jax.ShapeDtypeStruct((_SEQ, _N_CLS, _CTX_DIM), jnp.float32),
        mesh=mesh,
        scratch_types=(
            [pltpu.VMEM((_CHUNK, _CTX_DIM), jnp.float32)] * _NBUF
            + [pltpu.VMEM((_CTX_ROWS, _CTX_DIM), jnp.float32),
               pltpu.VMEM((_NU * _CHUNK,), jnp.int32)]
            + [pltpu.SemaphoreType.DMA] * (2 * _NBUF + 1)
        ),
        compiler_params=pltpu.CompilerParams(use_tc_tiling_on_sc=True),
    )(tokg, ctx32, token_embedding)


def kernel(tokenized_prompts, ctx, token_embedding):
    tokg = jnp.concatenate(
        [tokenized_prompts[:, :1], tokenized_prompts[:, 1 + _N_CTX:]], axis=1)
    tokg = tokg.T.reshape(-1)
    ctx32 = jnp.repeat(ctx, _CTX_ROWS, axis=0)
    out = _assemble(tokg, ctx32, token_embedding)
    return jnp.transpose(out, (1, 0, 2)), tokenized_prompts


# speedup vs baseline: 1.0036x; 1.0036x over previous
import functools

import jax
import jax.numpy as jnp
from jax import lax
from jax.experimental import pallas as pl
from jax.experimental.pallas import tpu as pltpu
from jax.experimental.pallas import tpu_sc as plsc

_N_CLS = 1024
_SEQ = 77
_N_CTX = 16
_CTX_DIM = 512
_NUM_CORES = 2
_NUM_SUBCORES = 16
_NW = _NUM_CORES * _NUM_SUBCORES
_CHUNK = 32
_NCHUNK = _N_CLS // _CHUNK
_NPOS = _SEQ - _N_CTX
_NGU = _NPOS * _NCHUNK
_NU = _NGU // _NW
_NBUF = 6
_CTX_ROWS = 32


def _assemble_body(tokg_hbm, ctx32_hbm, table_hbm, out_hbm,
                   st0, st1, st2, st3, st4, st5, ctx_rep, idx_all,
                   g0, g1, g2, g3, g4, g5, s0, s1, s2, s3, s4, s5, csem):
    wid = lax.axis_index("s") * _NUM_CORES + lax.axis_index("c")
    base = wid * _NU

    stages = (st0, st1, st2, st3, st4, st5)
    gsems = (g0, g1, g2, g3, g4, g5)
    ssems = (s0, s1, s2, s3, s4, s5)

    def unit_out(u):
        q = u // _NCHUNK
        t = jnp.where(q == 0, 0, q + _N_CTX)
        c0 = (u % _NCHUNK) * _CHUNK
        return t, c0

    def fire_gather(i, k):
        pltpu.async_copy(
            table_hbm.at[idx_all.at[pl.ds(i * _CHUNK, _CHUNK)]],
            stages[k], gsems[k])

    def wait_gather(k):
        pltpu.make_async_copy(
            table_hbm.at[pl.ds(0, _CHUNK)], stages[k], gsems[k]).wait()

    def fire_store(i, k):
        t, c0 = unit_out(base + i)
        pltpu.async_copy(
            stages[k], out_hbm.at[t, pl.ds(c0, _CHUNK)], ssems[k])

    def wait_store(k):
        pltpu.make_async_copy(
            stages[k], out_hbm.at[0, pl.ds(0, _CHUNK)], ssems[k]).wait()

    pltpu.sync_copy(tokg_hbm.at[pl.ds(base * _CHUNK, _NU * _CHUNK)], idx_all)
    fire_gather(0, 0)
    fire_gather(1, 1)

    tctx = wid // 2
    cbase = (wid % 2) * (_N_CLS // 2)
    pltpu.sync_copy(
        ctx32_hbm.at[pl.ds(tctx * _CTX_ROWS, _CTX_ROWS)], ctx_rep)
    for k in range(_N_CLS // 2 // _CTX_ROWS):
        pltpu.async_copy(
            ctx_rep,
            out_hbm.at[tctx + 1, pl.ds(cbase + k * _CTX_ROWS, _CTX_ROWS)],
            csem)

    def body(ip, carry):
        for k in range(_NBUF):
            u = _NBUF * ip + k

            @pl.when(u + 2 < _NU)
            def _(u=u, k=k):
                kn = (k + 2) % _NBUF
                if k >= 4:
                    wait_store(kn)
                else:
                    @pl.when(ip > 0)
                    def _():
                        wait_store(kn)
                fire_gather(u + 2, kn)

            @pl.when(u < _NU)
            def _(u=u, k=k):
                wait_gather(k)
                fire_store(u, k)
        return carry

    lax.fori_loop(0, -(-_NU // _NBUF), body, 0)
    for k in range(_NBUF):
        wait_store(k)
    for _k in range(_N_CLS // 2 // _CTX_ROWS):
        pltpu.make_async_copy(
            ctx_rep, out_hbm.at[0, pl.ds(0, _CTX_ROWS)], csem).wait()


@jax.jit
def _assemble(tokg, ctx32, token_embedding):
    mesh = plsc.VectorSubcoreMesh(
        core_axis_name="c", subcore_axis_name="s",
        num_cores=_NUM_CORES, num_subcores=_NUM_SUBCORES)
    return pl.kernel(
        _assemble_body,
        out_type=jax.ShapeDtypeStruct((_SEQ, _N_CLS, _CTX_DIM), jnp.float32),
        mesh=mesh,
        scratch_types=(
            [pltpu.VMEM((_CHUNK, _CTX_DIM), jnp.float32)] * _NBUF
            + [pltpu.VMEM((_CTX_ROWS, _CTX_DIM), jnp.float32),
               pltpu.VMEM((_NU * _CHUNK,), jnp.int32)]
            + [pltpu.SemaphoreType.DMA] * (2 * _NBUF + 1)
        ),
        compiler_params=pltpu.CompilerParams(use_tc_tiling_on_sc=True),
    )(tokg, ctx32, token_embedding)


def kernel(tokenized_prompts, ctx, token_embedding):
    tokt = tokenized_prompts.T
    tokg = jnp.concatenate([tokt[:1], tokt[1 + _N_CTX:]], axis=0).reshape(-1)
    ctx32 = jnp.repeat(ctx, _CTX_ROWS, axis=0)
    out = _assemble(tokg, ctx32, token_embedding)
    return jnp.transpose(out, (1, 0, 2)), tokenized_prompts
